# baseline (speedup 1.0000x reference)
import functools

import jax
import jax.numpy as jnp
from jax import lax
from jax.experimental import pallas as pl
from jax.experimental.pallas import tpu as pltpu
from jax.experimental.pallas import tpu_sc as plsc

N = 10000
E = 320000
D = 128

NQ = 4
DQ = D // NQ
NC = 2
NS = 16
EPW = E // NS
CB = 80
NCH = EPW // CB
K = 5
NGRP = NCH // K
NGRP2 = NGRP // 2
NZS = 10
RPS = N // NZS
ZB = 125

_mesh = plsc.VectorSubcoreMesh(
    core_axis_name="c", subcore_axis_name="s", num_cores=NC)
_sc_params = pltpu.CompilerParams(
    needs_layout_passes=False, use_tc_tiling_on_sc=False)


EPD = E // (NC * NS)


@functools.partial(
    pl.kernel,
    out_type=jax.ShapeDtypeStruct((NC * NS, N), jnp.float32),
    mesh=_mesh,
    compiler_params=_sc_params,
    scratch_types=[
        pltpu.VMEM((EPD,), jnp.int32),
        pltpu.VMEM((N,), jnp.float32),
    ],
)
def _deg_kernel(edge_hbm, out_hbm, dstv, dtile):
    c = lax.axis_index("c")
    s = lax.axis_index("s")
    wid = c * NS + s
    pltpu.sync_copy(edge_hbm.at[1, pl.ds(wid * EPD, EPD)], dstv)
    z16 = jnp.zeros((16,), jnp.float32)

    def zbody(i, carry):
        dtile[pl.ds(i * 16, 16)] = z16
        return carry

    lax.fori_loop(0, N // 16, zbody, 0)
    ones = jnp.ones((16,), jnp.float32)

    def body(i, carry):
        idx = dstv[pl.ds(i * 16, 16)]
        plsc.addupdate_scatter(dtile, [idx], ones)
        return carry

    lax.fori_loop(0, EPD // 16, body, 0)
    pltpu.sync_copy(dtile, out_hbm.at[wid])


@functools.partial(
    pl.kernel,
    out_type=jax.ShapeDtypeStruct((NQ, N, DQ), jnp.float32),
    mesh=_mesh,
    compiler_params=_sc_params,
    scratch_types=[
        pltpu.VMEM((EPW,), jnp.int32),
        pltpu.VMEM((EPW,), jnp.int32),
        pltpu.VMEM((2, K, CB, DQ), jnp.float32),
        pltpu.VMEM((ZB, DQ), jnp.float32),
        pltpu.VMEM_SHARED((N, DQ), jnp.float32),
        pltpu.SemaphoreType.DMA,
        pltpu.SemaphoreType.DMA,
    ],
)
def _agg_kernel(hp_hbm, edge_hbm, out_hbm,
                srcv, dstv, rows, zbuf, acc, semA, semB):
    c = lax.axis_index("c")
    s = lax.axis_index("s")
    pltpu.sync_copy(edge_hbm.at[0, pl.ds(s * EPW, EPW)], srcv)
    pltpu.sync_copy(edge_hbm.at[1, pl.ds(s * EPW, EPW)], dstv)

    @pl.when(s < NZS)
    def _zfill():
        z16 = jnp.zeros((16,), jnp.float32)

        def zbody(i, carry):
            zbuf[i, pl.ds(0, 16)] = z16
            zbuf[i, pl.ds(16, 16)] = z16
            return carry

        lax.fori_loop(0, ZB, zbody, 0)

    def issue(q, g, par, sem):
        for b in range(K):
            pltpu.async_copy(
                hp_hbm.at[q].at[srcv.at[pl.ds((g * K + b) * CB, CB)]],
                rows.at[par, b], sem)

    def drain_scatter(q, g, par, sem):
        for b in range(K):
            pltpu.make_async_copy(
                hp_hbm.at[q, pl.ds(0, CB)], rows.at[par, b], sem).wait()
        for b in range(K):
            pltpu.sync_copy(
                rows.at[par, b],
                acc.at[dstv.at[pl.ds((g * K + b) * CB, CB)]], add=True)

    for p in range(2):
        q = 2 * c + p

        issue(q, 0, 0, semA)

        @pl.when(s < NZS)
        def _zero():
            for j in range(RPS // ZB):
                pltpu.sync_copy(
                    zbuf, acc.at[pl.ds(s * RPS + j * ZB, ZB)])

        plsc.subcore_barrier()

        def body(i, carry):
            issue(q, 2 * i + 1, 1, semB)
            drain_scatter(q, 2 * i, 0, semA)

            @pl.when(i + 1 < NGRP2)
            def _next():
                issue(q, 2 * i + 2, 0, semA)

            drain_scatter(q, 2 * i + 1, 1, semB)
            return carry

        lax.fori_loop(0, NGRP2, body, 0)
        plsc.subcore_barrier()

        @pl.when(s < NZS)
        def _writeout():
            pltpu.sync_copy(acc.at[pl.ds(s * RPS, RPS)],
                            out_hbm.at[q, pl.ds(s * RPS, RPS)])

        plsc.subcore_barrier()


_BLK = 400
_GRID = N // _BLK


def _tc_pre(x, W1, degt):
    def body(xr, wr, dr, hpr, dinvr):
        deg = jnp.sum(dr[...], axis=1, keepdims=True) + 1.0
        dinv = lax.rsqrt(deg)
        h = jnp.dot(xr[...], wr[...], preferred_element_type=jnp.float32)
        hp = h * dinv
        for q in range(NQ):
            hpr[q] = hp[:, q * DQ:(q + 1) * DQ]
        dinvr[...] = dinv

    return pl.pallas_call(
        body,
        grid=(_GRID,),
        in_specs=[
            pl.BlockSpec((_BLK, D), lambda i: (i, 0)),
            pl.BlockSpec((D, D), lambda i: (0, 0)),
            pl.BlockSpec((_BLK, NC * NS), lambda i: (i, 0)),
        ],
        out_specs=[
            pl.BlockSpec((NQ, _BLK, DQ), lambda i: (0, i, 0)),
            pl.BlockSpec((_BLK, 1), lambda i: (i, 0)),
        ],
        out_shape=[
            jax.ShapeDtypeStruct((NQ, N, DQ), jnp.float32),
            jax.ShapeDtypeStruct((N, 1), jnp.float32),
        ],
    )(x, W1, degt)


def _tc_mid(agg, hp1, dinv, b1, W2):
    def body(ar, hr, dr, br, wr, outr):
        a = ar[...]
        hp = hr[...]
        s = jnp.concatenate(
            [a[q] + hp[q] for q in range(NQ)], axis=1)
        z = s * dr[...] + br[...]
        h = jnp.maximum(z, 0.0)
        hp2 = jnp.dot(h, wr[...],
                      preferred_element_type=jnp.float32) * dr[...]
        for q in range(NQ):
            outr[q] = hp2[:, q * DQ:(q + 1) * DQ]

    return pl.pallas_call(
        body,
        grid=(_GRID,),
        in_specs=[
            pl.BlockSpec((NQ, _BLK, DQ), lambda i: (0, i, 0)),
            pl.BlockSpec((NQ, _BLK, DQ), lambda i: (0, i, 0)),
            pl.BlockSpec((_BLK, 1), lambda i: (i, 0)),
            pl.BlockSpec((1, D), lambda i: (0, 0)),
            pl.BlockSpec((D, D), lambda i: (0, 0)),
        ],
        out_specs=pl.BlockSpec((NQ, _BLK, DQ), lambda i: (0, i, 0)),
        out_shape=jax.ShapeDtypeStruct((NQ, N, DQ), jnp.float32),
    )(agg, hp1, dinv, b1, W2)


def _tc_fin(agg, hp2, dinv, b2):
    def body(ar, hr, dr, br, outr):
        a = ar[...]
        hp = hr[...]
        s = jnp.concatenate(
            [a[q] + hp[q] for q in range(NQ)], axis=1)
        outr[...] = s * dr[...] + br[...]

    return pl.pallas_call(
        body,
        grid=(_GRID,),
        in_specs=[
            pl.BlockSpec((NQ, _BLK, DQ), lambda i: (0, i, 0)),
            pl.BlockSpec((NQ, _BLK, DQ), lambda i: (0, i, 0)),
            pl.BlockSpec((_BLK, 1), lambda i: (i, 0)),
            pl.BlockSpec((1, D), lambda i: (0, 0)),
        ],
        out_specs=pl.BlockSpec((_BLK, D), lambda i: (i, 0)),
        out_shape=jax.ShapeDtypeStruct((N, D), jnp.float32),
    )(agg, hp2, dinv, b2)


def kernel(x, edge_index, W1, b1, W2, b2):
    degp = _deg_kernel(edge_index)
    degt = degp.T
    hp1, dinv = _tc_pre(x, W1, degt)
    agg1 = _agg_kernel(hp1, edge_index)
    hp2 = _tc_mid(agg1, hp1, dinv, b1.reshape(1, D), W2)
    agg2 = _agg_kernel(hp2, edge_index)
    return _tc_fin(agg2, hp2, dinv, b2.reshape(1, D))


# speedup vs baseline: 1.0373x; 1.0373x over previous
import functools

import jax
import jax.numpy as jnp
from jax import lax
from jax.experimental import pallas as pl
from jax.experimental.pallas import tpu as pltpu
from jax.experimental.pallas import tpu_sc as plsc

N = 10000
E = 320000
D = 128

NQ = 4
DQ = D // NQ
NC = 2
NS = 16
EPW = E // NS
CB = 125
NCH = EPW // CB
K = 5
NGRP = NCH // K
NGRP2 = NGRP // 2
NZS = 10
RPS = N // NZS
ZB = 125

_mesh = plsc.VectorSubcoreMesh(
    core_axis_name="c", subcore_axis_name="s", num_cores=NC)
_sc_params = pltpu.CompilerParams(
    needs_layout_passes=False, use_tc_tiling_on_sc=False)


EPD = E // (NC * NS)


@functools.partial(
    pl.kernel,
    out_type=jax.ShapeDtypeStruct((NC * NS, N), jnp.float32),
    mesh=_mesh,
    compiler_params=_sc_params,
    scratch_types=[
        pltpu.VMEM((EPD,), jnp.int32),
        pltpu.VMEM((N,), jnp.float32),
    ],
)
def _deg_kernel(edge_hbm, out_hbm, dstv, dtile):
    c = lax.axis_index("c")
    s = lax.axis_index("s")
    wid = c * NS + s
    pltpu.sync_copy(edge_hbm.at[1, pl.ds(wid * EPD, EPD)], dstv)
    z16 = jnp.zeros((16,), jnp.float32)

    def zbody(i, carry):
        dtile[pl.ds(i * 16, 16)] = z16
        return carry

    lax.fori_loop(0, N // 16, zbody, 0)
    ones = jnp.ones((16,), jnp.float32)

    def body(i, carry):
        idx = dstv[pl.ds(i * 16, 16)]
        plsc.addupdate_scatter(dtile, [idx], ones)
        return carry

    lax.fori_loop(0, EPD // 16, body, 0)
    pltpu.sync_copy(dtile, out_hbm.at[wid])


@functools.partial(
    pl.kernel,
    out_type=jax.ShapeDtypeStruct((NQ, N, DQ), jnp.float32),
    mesh=_mesh,
    compiler_params=_sc_params,
    scratch_types=[
        pltpu.VMEM((NCH, CB), jnp.int32),
        pltpu.VMEM((NCH, CB), jnp.int32),
        pltpu.VMEM((2, K, CB, DQ), jnp.float32),
        pltpu.VMEM((ZB, DQ), jnp.float32),
        pltpu.VMEM_SHARED((N, DQ), jnp.float32),
        pltpu.SemaphoreType.DMA,
        pltpu.SemaphoreType.DMA,
    ],
)
def _agg_kernel(hp_hbm, src_hbm, dst_hbm, out_hbm,
                srcv, dstv, rows, zbuf, acc, semA, semB):
    c = lax.axis_index("c")
    s = lax.axis_index("s")
    pltpu.sync_copy(src_hbm.at[s], srcv)
    pltpu.sync_copy(dst_hbm.at[s], dstv)

    @pl.when(s < NZS)
    def _zfill():
        z16 = jnp.zeros((16,), jnp.float32)

        def zbody(i, carry):
            zbuf[i, pl.ds(0, 16)] = z16
            zbuf[i, pl.ds(16, 16)] = z16
            return carry

        lax.fori_loop(0, ZB, zbody, 0)

    def issue(q, g, par, sem):
        for b in range(K):
            pltpu.async_copy(
                hp_hbm.at[q].at[srcv.at[g * K + b]],
                rows.at[par, b], sem)

    def drain_scatter(q, g, par, sem):
        for b in range(K):
            pltpu.make_async_copy(
                hp_hbm.at[q, pl.ds(0, CB)], rows.at[par, b], sem).wait()
        for b in range(K):
            pltpu.sync_copy(
                rows.at[par, b],
                acc.at[dstv.at[g * K + b]], add=True)

    for p in range(2):
        q = 2 * c + p

        issue(q, 0, 0, semA)

        @pl.when(s < NZS)
        def _zero():
            for j in range(RPS // ZB):
                pltpu.sync_copy(
                    zbuf, acc.at[pl.ds(s * RPS + j * ZB, ZB)])

        plsc.subcore_barrier()

        def body(i, carry):
            issue(q, 2 * i + 1, 1, semB)
            drain_scatter(q, 2 * i, 0, semA)

            @pl.when(i + 1 < NGRP2)
            def _next():
                issue(q, 2 * i + 2, 0, semA)

            drain_scatter(q, 2 * i + 1, 1, semB)
            return carry

        lax.fori_loop(0, NGRP2, body, 0)
        plsc.subcore_barrier()

        @pl.when(s < NZS)
        def _writeout():
            pltpu.sync_copy(acc.at[pl.ds(s * RPS, RPS)],
                            out_hbm.at[q, pl.ds(s * RPS, RPS)])

        plsc.subcore_barrier()


_BLK = 400
_GRID = N // _BLK


def _tc_pre(x, W1, degt):
    def body(xr, wr, dr, hpr, dinvr):
        deg = jnp.sum(dr[...], axis=1, keepdims=True) + 1.0
        dinv = lax.rsqrt(deg)
        h = jnp.dot(xr[...], wr[...], preferred_element_type=jnp.float32)
        hp = h * dinv
        for q in range(NQ):
            hpr[q] = hp[:, q * DQ:(q + 1) * DQ]
        dinvr[...] = dinv

    return pl.pallas_call(
        body,
        grid=(_GRID,),
        in_specs=[
            pl.BlockSpec((_BLK, D), lambda i: (i, 0)),
            pl.BlockSpec((D, D), lambda i: (0, 0)),
            pl.BlockSpec((_BLK, NC * NS), lambda i: (i, 0)),
        ],
        out_specs=[
            pl.BlockSpec((NQ, _BLK, DQ), lambda i: (0, i, 0)),
            pl.BlockSpec((_BLK, 1), lambda i: (i, 0)),
        ],
        out_shape=[
            jax.ShapeDtypeStruct((NQ, N, DQ), jnp.float32),
            jax.ShapeDtypeStruct((N, 1), jnp.float32),
        ],
    )(x, W1, degt)


def _tc_mid(agg, hp1, dinv, b1, W2):
    def body(ar, hr, dr, br, wr, outr):
        a = ar[...]
        hp = hr[...]
        s = jnp.concatenate(
            [a[q] + hp[q] for q in range(NQ)], axis=1)
        z = s * dr[...] + br[...]
        h = jnp.maximum(z, 0.0)
        hp2 = jnp.dot(h, wr[...],
                      preferred_element_type=jnp.float32) * dr[...]
        for q in range(NQ):
            outr[q] = hp2[:, q * DQ:(q + 1) * DQ]

    return pl.pallas_call(
        body,
        grid=(_GRID,),
        in_specs=[
            pl.BlockSpec((NQ, _BLK, DQ), lambda i: (0, i, 0)),
            pl.BlockSpec((NQ, _BLK, DQ), lambda i: (0, i, 0)),
            pl.BlockSpec((_BLK, 1), lambda i: (i, 0)),
            pl.BlockSpec((1, D), lambda i: (0, 0)),
            pl.BlockSpec((D, D), lambda i: (0, 0)),
        ],
        out_specs=pl.BlockSpec((NQ, _BLK, DQ), lambda i: (0, i, 0)),
        out_shape=jax.ShapeDtypeStruct((NQ, N, DQ), jnp.float32),
    )(agg, hp1, dinv, b1, W2)


def _tc_fin(agg, hp2, dinv, b2):
    def body(ar, hr, dr, br, outr):
        a = ar[...]
        hp = hr[...]
        s = jnp.concatenate(
            [a[q] + hp[q] for q in range(NQ)], axis=1)
        outr[...] = s * dr[...] + br[...]

    return pl.pallas_call(
        body,
        grid=(_GRID,),
        in_specs=[
            pl.BlockSpec((NQ, _BLK, DQ), lambda i: (0, i, 0)),
            pl.BlockSpec((NQ, _BLK, DQ), lambda i: (0, i, 0)),
            pl.BlockSpec((_BLK, 1), lambda i: (i, 0)),
            pl.BlockSpec((1, D), lambda i: (0, 0)),
        ],
        out_specs=pl.BlockSpec((_BLK, D), lambda i: (i, 0)),
        out_shape=jax.ShapeDtypeStruct((N, D), jnp.float32),
    )(agg, hp2, dinv, b2)


def kernel(x, edge_index, W1, b1, W2, b2):
    src = edge_index[0].reshape(NS, NCH, CB)
    dst = edge_index[1].reshape(NS, NCH, CB)
    degp = _deg_kernel(edge_index)
    degt = degp.T
    hp1, dinv = _tc_pre(x, W1, degt)
    agg1 = _agg_kernel(hp1, src, dst)
    hp2 = _tc_mid(agg1, hp1, dinv, b1.reshape(1, D), W2)
    agg2 = _agg_kernel(hp2, src, dst)
    return _tc_fin(agg2, hp2, dinv, b2.reshape(1, D))


# speedup vs baseline: 1.2051x; 1.1618x over previous
import functools

import jax
import jax.numpy as jnp
from jax import lax
from jax.experimental import pallas as pl
from jax.experimental.pallas import tpu as pltpu
from jax.experimental.pallas import tpu_sc as plsc

N = 10000
E = 320000
D = 128

NQ = 4
DQ = D // NQ
NC = 2
NS = 16
EPW = E // NS
CB = 80
NCH = EPW // CB
K = 5
NGRP = NCH // K
NGRP2 = NGRP // 2
NZS = 10
RPS = N // NZS
ZB = 125

_mesh = plsc.VectorSubcoreMesh(
    core_axis_name="c", subcore_axis_name="s", num_cores=NC)
_sc_params = pltpu.CompilerParams(
    needs_layout_passes=False, use_tc_tiling_on_sc=False)


EPD = E // (NC * NS)


@functools.partial(
    pl.kernel,
    out_type=jax.ShapeDtypeStruct((NC * NS, N), jnp.float32),
    mesh=_mesh,
    compiler_params=_sc_params,
    scratch_types=[
        pltpu.VMEM((EPD,), jnp.int32),
        pltpu.VMEM((N,), jnp.float32),
    ],
)
def _deg_kernel(edge_hbm, out_hbm, dstv, dtile):
    c = lax.axis_index("c")
    s = lax.axis_index("s")
    wid = c * NS + s
    pltpu.sync_copy(edge_hbm.at[1, pl.ds(wid * EPD, EPD)], dstv)
    z16 = jnp.zeros((16,), jnp.float32)

    def zbody(i, carry):
        dtile[pl.ds(i * 16, 16)] = z16
        return carry

    lax.fori_loop(0, N // 16, zbody, 0)
    ones = jnp.ones((16,), jnp.float32)

    def body(i, carry):
        idx = dstv[pl.ds(i * 16, 16)]
        plsc.addupdate_scatter(dtile, [idx], ones)
        return carry

    lax.fori_loop(0, EPD // 16, body, 0)
    pltpu.sync_copy(dtile, out_hbm.at[wid])


@functools.partial(
    pl.kernel,
    out_type=jax.ShapeDtypeStruct((N, D), jnp.float32),
    mesh=_mesh,
    compiler_params=_sc_params,
    scratch_types=[
        pltpu.VMEM((NCH, CB), jnp.int32),
        pltpu.VMEM((NCH, CB), jnp.int32),
        pltpu.VMEM((2, K, CB, DQ), jnp.float32),
        pltpu.VMEM((ZB, DQ), jnp.float32),
        pltpu.VMEM_SHARED((N, DQ), jnp.float32),
        pltpu.SemaphoreType.DMA,
        pltpu.SemaphoreType.DMA,
    ],
)
def _agg_kernel(hp_hbm, src_hbm, dst_hbm, out_hbm,
                srcv, dstv, rows, zbuf, acc, semA, semB):
    c = lax.axis_index("c")
    s = lax.axis_index("s")
    pltpu.sync_copy(src_hbm.at[s], srcv)
    pltpu.sync_copy(dst_hbm.at[s], dstv)

    @pl.when(s < NZS)
    def _zfill():
        z16 = jnp.zeros((16,), jnp.float32)

        def zbody(i, carry):
            zbuf[i, pl.ds(0, 16)] = z16
            zbuf[i, pl.ds(16, 16)] = z16
            return carry

        lax.fori_loop(0, ZB, zbody, 0)

    def issue(g, par, sem):
        for b in range(K):
            pltpu.async_copy(
                hp_hbm.at[srcv.at[g * K + b]], rows.at[par, b], sem)

    def drain_scatter(g, par, sem):
        for b in range(K):
            pltpu.make_async_copy(
                hp_hbm.at[pl.ds(0, CB)], rows.at[par, b], sem).wait()
        for b in range(K):
            pltpu.sync_copy(
                rows.at[par, b], acc.at[dstv.at[g * K + b]], add=True)

    def shift_idx(mul, add):
        def sbody(i, carry):
            for k in range(CB // 16):
                sl = pl.ds(k * 16, 16)
                srcv[i, sl] = srcv[i, sl] * mul + add
            return carry

        lax.fori_loop(0, NCH, sbody, 0)

    for p in range(2):
        q = 2 * c + p

        if p == 0:
            shift_idx(NQ, q)
        else:
            shift_idx(1, 1)

        issue(0, 0, semA)

        @pl.when(s < NZS)
        def _zero():
            for j in range(RPS // ZB):
                pltpu.sync_copy(
                    zbuf, acc.at[pl.ds(s * RPS + j * ZB, ZB)])

        plsc.subcore_barrier()

        def body(i, carry):
            issue(2 * i + 1, 1, semB)
            drain_scatter(2 * i, 0, semA)

            @pl.when(i + 1 < NGRP2)
            def _next():
                issue(2 * i + 2, 0, semA)

            drain_scatter(2 * i + 1, 1, semB)
            return carry

        lax.fori_loop(0, NGRP2, body, 0)
        plsc.subcore_barrier()

        @pl.when(s < NZS)
        def _writeout():
            pltpu.sync_copy(
                acc.at[pl.ds(s * RPS, RPS)],
                out_hbm.at[pl.ds(s * RPS, RPS), pl.ds(q * DQ, DQ)])

        plsc.subcore_barrier()


_BLK = 400
_GRID = N // _BLK


def _tc_pre(x, W1, degt):
    def body(xr, wr, dr, hpr, dinvr):
        deg = jnp.sum(dr[...], axis=1, keepdims=True) + 1.0
        dinv = lax.rsqrt(deg)
        h = jnp.dot(xr[...], wr[...], preferred_element_type=jnp.float32)
        hpr[...] = h * dinv
        dinvr[...] = dinv

    return pl.pallas_call(
        body,
        grid=(_GRID,),
        in_specs=[
            pl.BlockSpec((_BLK, D), lambda i: (i, 0)),
            pl.BlockSpec((D, D), lambda i: (0, 0)),
            pl.BlockSpec((_BLK, NC * NS), lambda i: (i, 0)),
        ],
        out_specs=[
            pl.BlockSpec((_BLK, D), lambda i: (i, 0)),
            pl.BlockSpec((_BLK, 1), lambda i: (i, 0)),
        ],
        out_shape=[
            jax.ShapeDtypeStruct((N, D), jnp.float32),
            jax.ShapeDtypeStruct((N, 1), jnp.float32),
        ],
    )(x, W1, degt)


def _tc_mid(agg, hp1, dinv, b1, W2):
    def body(ar, hr, dr, br, wr, outr):
        z = (ar[...] + hr[...]) * dr[...] + br[...]
        h = jnp.maximum(z, 0.0)
        outr[...] = jnp.dot(h, wr[...],
                            preferred_element_type=jnp.float32) * dr[...]

    return pl.pallas_call(
        body,
        grid=(_GRID,),
        in_specs=[
            pl.BlockSpec((_BLK, D), lambda i: (i, 0)),
            pl.BlockSpec((_BLK, D), lambda i: (i, 0)),
            pl.BlockSpec((_BLK, 1), lambda i: (i, 0)),
            pl.BlockSpec((1, D), lambda i: (0, 0)),
            pl.BlockSpec((D, D), lambda i: (0, 0)),
        ],
        out_specs=pl.BlockSpec((_BLK, D), lambda i: (i, 0)),
        out_shape=jax.ShapeDtypeStruct((N, D), jnp.float32),
    )(agg, hp1, dinv, b1, W2)


def _tc_fin(agg, hp2, dinv, b2):
    def body(ar, hr, dr, br, outr):
        outr[...] = (ar[...] + hr[...]) * dr[...] + br[...]

    return pl.pallas_call(
        body,
        grid=(_GRID,),
        in_specs=[
            pl.BlockSpec((_BLK, D), lambda i: (i, 0)),
            pl.BlockSpec((_BLK, D), lambda i: (i, 0)),
            pl.BlockSpec((_BLK, 1), lambda i: (i, 0)),
            pl.BlockSpec((1, D), lambda i: (0, 0)),
        ],
        out_specs=pl.BlockSpec((_BLK, D), lambda i: (i, 0)),
        out_shape=jax.ShapeDtypeStruct((N, D), jnp.float32),
    )(agg, hp2, dinv, b2)


def kernel(x, edge_index, W1, b1, W2, b2):
    src = edge_index[0].reshape(NS, NCH, CB)
    dst = edge_index[1].reshape(NS, NCH, CB)
    degp = _deg_kernel(edge_index)
    degt = degp.T
    hp1, dinv = _tc_pre(x, W1, degt)
    agg1 = _agg_kernel(hp1.reshape(NQ * N, DQ), src, dst)
    hp2 = _tc_mid(agg1, hp1, dinv, b1.reshape(1, D), W2)
    agg2 = _agg_kernel(hp2.reshape(NQ * N, DQ), src, dst)
    return _tc_fin(agg2, hp2, dinv, b2.reshape(1, D))


# speedup vs baseline: 1.3686x; 1.1356x over previous
import functools

import jax
import jax.numpy as jnp
from jax import lax
from jax.experimental import pallas as pl
from jax.experimental.pallas import tpu as pltpu
from jax.experimental.pallas import tpu_sc as plsc

N = 10000
E = 320000
D = 128

NQ = 4
DQ = D // NQ
NC = 2
NS = 16
EPW = E // NS
CB = 125
NCH = EPW // CB
K = 5
NGRP = NCH // K
NGRP2 = NGRP // 2
NZS = 10
RPS = N // NZS
ZB = 125

_mesh = plsc.VectorSubcoreMesh(
    core_axis_name="c", subcore_axis_name="s", num_cores=NC)
_sc_params = pltpu.CompilerParams(
    needs_layout_passes=False, use_tc_tiling_on_sc=False)


EPD = E // (NC * NS)


@functools.partial(
    pl.kernel,
    out_type=jax.ShapeDtypeStruct((NC * NS, N), jnp.float32),
    mesh=_mesh,
    compiler_params=_sc_params,
    scratch_types=[
        pltpu.VMEM((EPD,), jnp.int32),
        pltpu.VMEM((N,), jnp.float32),
    ],
)
def _deg_kernel(edge_hbm, out_hbm, dstv, dtile):
    c = lax.axis_index("c")
    s = lax.axis_index("s")
    wid = c * NS + s
    pltpu.sync_copy(edge_hbm.at[1, pl.ds(wid * EPD, EPD)], dstv)
    z16 = jnp.zeros((16,), jnp.float32)

    def zbody(i, carry):
        dtile[pl.ds(i * 16, 16)] = z16
        return carry

    lax.fori_loop(0, N // 16, zbody, 0)
    ones = jnp.ones((16,), jnp.float32)

    def body(i, carry):
        idx = dstv[pl.ds(i * 16, 16)]
        plsc.addupdate_scatter(dtile, [idx], ones)
        return carry

    lax.fori_loop(0, EPD // 16, body, 0)
    pltpu.sync_copy(dtile, out_hbm.at[wid])


@functools.partial(
    pl.kernel,
    out_type=jax.ShapeDtypeStruct((N, D), jnp.float32),
    mesh=_mesh,
    compiler_params=_sc_params,
    scratch_types=[
        pltpu.VMEM((NCH, CB), jnp.int32),
        pltpu.VMEM((NCH, CB), jnp.int32),
        pltpu.VMEM((2, K, CB, DQ), jnp.float32),
        pltpu.VMEM((ZB, DQ), jnp.float32),
        pltpu.VMEM_SHARED((N, DQ), jnp.float32),
        pltpu.SemaphoreType.DMA,
        pltpu.SemaphoreType.DMA,
    ],
)
def _agg_kernel(hp_hbm, src4_hbm, dst_hbm, out_hbm,
                srcv, dstv, rows, zbuf, acc, semA, semB):
    c = lax.axis_index("c")
    s = lax.axis_index("s")
    pltpu.sync_copy(dst_hbm.at[s], dstv)

    @pl.when(s < NZS)
    def _zfill():
        z16 = jnp.zeros((16,), jnp.float32)

        def zbody(i, carry):
            zbuf[i, pl.ds(0, 16)] = z16
            zbuf[i, pl.ds(16, 16)] = z16
            return carry

        lax.fori_loop(0, ZB, zbody, 0)

    def issue(g, par, sem):
        for b in range(K):
            pltpu.async_copy(
                hp_hbm.at[srcv.at[g * K + b]], rows.at[par, b], sem)

    def drain_scatter(g, par, sem):
        for b in range(K):
            pltpu.make_async_copy(
                hp_hbm.at[pl.ds(0, CB)], rows.at[par, b], sem).wait()
        for b in range(K):
            pltpu.sync_copy(
                rows.at[par, b], acc.at[dstv.at[g * K + b]], add=True)

    for p in range(2):
        q = 2 * c + p
        pltpu.sync_copy(src4_hbm.at[q, s], srcv)

        issue(0, 0, semA)

        @pl.when(s < NZS)
        def _zero():
            for j in range(RPS // ZB):
                pltpu.sync_copy(
                    zbuf, acc.at[pl.ds(s * RPS + j * ZB, ZB)])

        plsc.subcore_barrier()

        def body(i, carry):
            issue(2 * i + 1, 1, semB)
            drain_scatter(2 * i, 0, semA)

            @pl.when(i + 1 < NGRP2)
            def _next():
                issue(2 * i + 2, 0, semA)

            drain_scatter(2 * i + 1, 1, semB)
            return carry

        lax.fori_loop(0, NGRP2, body, 0)
        plsc.subcore_barrier()

        @pl.when(s < NZS)
        def _writeout():
            pltpu.sync_copy(
                acc.at[pl.ds(s * RPS, RPS)],
                out_hbm.at[pl.ds(s * RPS, RPS), pl.ds(q * DQ, DQ)])

        plsc.subcore_barrier()


_BLK = 1000
_GRID = N // _BLK


def _tc_pre(x, W1, degt):
    def body(xr, wr, dr, hpr, dinvr):
        deg = jnp.sum(dr[...], axis=1, keepdims=True) + 1.0
        dinv = lax.rsqrt(deg)
        h = jnp.dot(xr[...], wr[...], preferred_element_type=jnp.float32)
        hpr[...] = h * dinv
        dinvr[...] = dinv

    return pl.pallas_call(
        body,
        grid=(_GRID,),
        in_specs=[
            pl.BlockSpec((_BLK, D), lambda i: (i, 0)),
            pl.BlockSpec((D, D), lambda i: (0, 0)),
            pl.BlockSpec((_BLK, NC * NS), lambda i: (i, 0)),
        ],
        out_specs=[
            pl.BlockSpec((_BLK, D), lambda i: (i, 0)),
            pl.BlockSpec((_BLK, 1), lambda i: (i, 0)),
        ],
        out_shape=[
            jax.ShapeDtypeStruct((N, D), jnp.float32),
            jax.ShapeDtypeStruct((N, 1), jnp.float32),
        ],
    )(x, W1, degt)


def _tc_mid(agg, hp1, dinv, b1, W2):
    def body(ar, hr, dr, br, wr, outr):
        z = (ar[...] + hr[...]) * dr[...] + br[...]
        h = jnp.maximum(z, 0.0)
        outr[...] = jnp.dot(h, wr[...],
                            preferred_element_type=jnp.float32) * dr[...]

    return pl.pallas_call(
        body,
        grid=(_GRID,),
        in_specs=[
            pl.BlockSpec((_BLK, D), lambda i: (i, 0)),
            pl.BlockSpec((_BLK, D), lambda i: (i, 0)),
            pl.BlockSpec((_BLK, 1), lambda i: (i, 0)),
            pl.BlockSpec((1, D), lambda i: (0, 0)),
            pl.BlockSpec((D, D), lambda i: (0, 0)),
        ],
        out_specs=pl.BlockSpec((_BLK, D), lambda i: (i, 0)),
        out_shape=jax.ShapeDtypeStruct((N, D), jnp.float32),
    )(agg, hp1, dinv, b1, W2)


def _tc_fin(agg, hp2, dinv, b2):
    def body(ar, hr, dr, br, outr):
        outr[...] = (ar[...] + hr[...]) * dr[...] + br[...]

    return pl.pallas_call(
        body,
        grid=(_GRID,),
        in_specs=[
            pl.BlockSpec((_BLK, D), lambda i: (i, 0)),
            pl.BlockSpec((_BLK, D), lambda i: (i, 0)),
            pl.BlockSpec((_BLK, 1), lambda i: (i, 0)),
            pl.BlockSpec((1, D), lambda i: (0, 0)),
        ],
        out_specs=pl.BlockSpec((_BLK, D), lambda i: (i, 0)),
        out_shape=jax.ShapeDtypeStruct((N, D), jnp.float32),
    )(agg, hp2, dinv, b2)


def kernel(x, edge_index, W1, b1, W2, b2):
    src4 = (edge_index[0] * NQ + jnp.arange(NQ, dtype=jnp.int32)[:, None]
            ).reshape(NQ, NS, NCH, CB)
    dst = edge_index[1].reshape(NS, NCH, CB)
    degp = _deg_kernel(edge_index)
    degt = degp.T
    hp1, dinv = _tc_pre(x, W1, degt)
    agg1 = _agg_kernel(hp1.reshape(NQ * N, DQ), src4, dst)
    hp2 = _tc_mid(agg1, hp1, dinv, b1.reshape(1, D), W2)
    agg2 = _agg_kernel(hp2.reshape(NQ * N, DQ), src4, dst)
    return _tc_fin(agg2, hp2, dinv, b2.reshape(1, D))


# speedup vs baseline: 1.3863x; 1.0129x over previous
import functools

import jax
import jax.numpy as jnp
from jax import lax
from jax.experimental import pallas as pl
from jax.experimental.pallas import tpu as pltpu
from jax.experimental.pallas import tpu_sc as plsc

N = 10000
E = 320000
D = 128

NQ = 4
DQ = D // NQ
NC = 2
NS = 16
EPW = E // NS
CB = 125
NCH = EPW // CB
K = 8
NGRP = NCH // K
NGRP2 = NGRP // 2
NZS = 10
RPS = N // NZS
ZB = 125

_mesh = plsc.VectorSubcoreMesh(
    core_axis_name="c", subcore_axis_name="s", num_cores=NC)
_sc_params = pltpu.CompilerParams(
    needs_layout_passes=False, use_tc_tiling_on_sc=False)


EPD = E // (NC * NS)


@functools.partial(
    pl.kernel,
    out_type=jax.ShapeDtypeStruct((NC * NS, N), jnp.float32),
    mesh=_mesh,
    compiler_params=_sc_params,
    scratch_types=[
        pltpu.VMEM((EPD,), jnp.int32),
        pltpu.VMEM((N,), jnp.float32),
    ],
)
def _deg_kernel(edge_hbm, out_hbm, dstv, dtile):
    c = lax.axis_index("c")
    s = lax.axis_index("s")
    wid = c * NS + s
    pltpu.sync_copy(edge_hbm.at[1, pl.ds(wid * EPD, EPD)], dstv)
    z16 = jnp.zeros((16,), jnp.float32)

    def zbody(i, carry):
        dtile[pl.ds(i * 16, 16)] = z16
        return carry

    lax.fori_loop(0, N // 16, zbody, 0)
    ones = jnp.ones((16,), jnp.float32)

    def body(i, carry):
        idx = dstv[pl.ds(i * 16, 16)]
        plsc.addupdate_scatter(dtile, [idx], ones)
        return carry

    lax.fori_loop(0, EPD // 16, body, 0)
    pltpu.sync_copy(dtile, out_hbm.at[wid])


@functools.partial(
    pl.kernel,
    out_type=jax.ShapeDtypeStruct((N, D), jnp.float32),
    mesh=_mesh,
    compiler_params=_sc_params,
    scratch_types=[
        pltpu.VMEM((NCH, CB), jnp.int32),
        pltpu.VMEM((NCH, CB), jnp.int32),
        pltpu.VMEM((2, K, CB, DQ), jnp.float32),
        pltpu.VMEM((ZB, DQ), jnp.float32),
        pltpu.VMEM_SHARED((N, DQ), jnp.float32),
        pltpu.SemaphoreType.DMA,
        pltpu.SemaphoreType.DMA,
    ],
)
def _agg_kernel(hp_hbm, src4_hbm, dst_hbm, out_hbm,
                srcv, dstv, rows, zbuf, acc, semA, semB):
    c = lax.axis_index("c")
    s = lax.axis_index("s")
    pltpu.sync_copy(dst_hbm.at[s], dstv)

    @pl.when(s < NZS)
    def _zfill():
        z16 = jnp.zeros((16,), jnp.float32)

        def zbody(i, carry):
            zbuf[i, pl.ds(0, 16)] = z16
            zbuf[i, pl.ds(16, 16)] = z16
            return carry

        lax.fori_loop(0, ZB, zbody, 0)

    def issue(g, par, sem):
        for b in range(K):
            pltpu.async_copy(
                hp_hbm.at[srcv.at[g * K + b]], rows.at[par, b], sem)

    def drain_scatter(g, par, sem):
        for b in range(K):
            pltpu.make_async_copy(
                hp_hbm.at[pl.ds(0, CB)], rows.at[par, b], sem).wait()
        for b in range(K):
            pltpu.sync_copy(
                rows.at[par, b], acc.at[dstv.at[g * K + b]], add=True)

    for p in range(2):
        q = 2 * c + p
        pltpu.sync_copy(src4_hbm.at[q, s], srcv)

        issue(0, 0, semA)

        @pl.when(s < NZS)
        def _zero():
            for j in range(RPS // ZB):
                pltpu.sync_copy(
                    zbuf, acc.at[pl.ds(s * RPS + j * ZB, ZB)])

        plsc.subcore_barrier()

        def body(i, carry):
            issue(2 * i + 1, 1, semB)
            drain_scatter(2 * i, 0, semA)

            @pl.when(i + 1 < NGRP2)
            def _next():
                issue(2 * i + 2, 0, semA)

            drain_scatter(2 * i + 1, 1, semB)
            return carry

        lax.fori_loop(0, NGRP2, body, 0)
        plsc.subcore_barrier()

        @pl.when(s < NZS)
        def _writeout():
            pltpu.sync_copy(
                acc.at[pl.ds(s * RPS, RPS)],
                out_hbm.at[pl.ds(s * RPS, RPS), pl.ds(q * DQ, DQ)])

        plsc.subcore_barrier()


_BLK = 1000
_GRID = N // _BLK


def _tc_pre(x, W1, degt):
    def body(xr, wr, dr, hpr, dinvr):
        deg = jnp.sum(dr[...], axis=1, keepdims=True) + 1.0
        dinv = lax.rsqrt(deg)
        h = jnp.dot(xr[...], wr[...], preferred_element_type=jnp.float32)
        hpr[...] = h * dinv
        dinvr[...] = dinv

    return pl.pallas_call(
        body,
        grid=(_GRID,),
        in_specs=[
            pl.BlockSpec((_BLK, D), lambda i: (i, 0)),
            pl.BlockSpec((D, D), lambda i: (0, 0)),
            pl.BlockSpec((_BLK, NC * NS), lambda i: (i, 0)),
        ],
        out_specs=[
            pl.BlockSpec((_BLK, D), lambda i: (i, 0)),
            pl.BlockSpec((_BLK, 1), lambda i: (i, 0)),
        ],
        out_shape=[
            jax.ShapeDtypeStruct((N, D), jnp.float32),
            jax.ShapeDtypeStruct((N, 1), jnp.float32),
        ],
    )(x, W1, degt)


def _tc_mid(agg, hp1, dinv, b1, W2):
    def body(ar, hr, dr, br, wr, outr):
        z = (ar[...] + hr[...]) * dr[...] + br[...]
        h = jnp.maximum(z, 0.0)
        outr[...] = jnp.dot(h, wr[...],
                            preferred_element_type=jnp.float32) * dr[...]

    return pl.pallas_call(
        body,
        grid=(_GRID,),
        in_specs=[
            pl.BlockSpec((_BLK, D), lambda i: (i, 0)),
            pl.BlockSpec((_BLK, D), lambda i: (i, 0)),
            pl.BlockSpec((_BLK, 1), lambda i: (i, 0)),
            pl.BlockSpec((1, D), lambda i: (0, 0)),
            pl.BlockSpec((D, D), lambda i: (0, 0)),
        ],
        out_specs=pl.BlockSpec((_BLK, D), lambda i: (i, 0)),
        out_shape=jax.ShapeDtypeStruct((N, D), jnp.float32),
    )(agg, hp1, dinv, b1, W2)


def _tc_fin(agg, hp2, dinv, b2):
    def body(ar, hr, dr, br, outr):
        outr[...] = (ar[...] + hr[...]) * dr[...] + br[...]

    return pl.pallas_call(
        body,
        grid=(_GRID,),
        in_specs=[
            pl.BlockSpec((_BLK, D), lambda i: (i, 0)),
            pl.BlockSpec((_BLK, D), lambda i: (i, 0)),
            pl.BlockSpec((_BLK, 1), lambda i: (i, 0)),
            pl.BlockSpec((1, D), lambda i: (0, 0)),
        ],
        out_specs=pl.BlockSpec((_BLK, D), lambda i: (i, 0)),
        out_shape=jax.ShapeDtypeStruct((N, D), jnp.float32),
    )(agg, hp2, dinv, b2)


def kernel(x, edge_index, W1, b1, W2, b2):
    src4 = (edge_index[0] * NQ + jnp.arange(NQ, dtype=jnp.int32)[:, None]
            ).reshape(NQ, NS, NCH, CB)
    dst = edge_index[1].reshape(NS, NCH, CB)
    degp = _deg_kernel(edge_index)
    degt = degp.T
    hp1, dinv = _tc_pre(x, W1, degt)
    agg1 = _agg_kernel(hp1.reshape(NQ * N, DQ), src4, dst)
    hp2 = _tc_mid(agg1, hp1, dinv, b1.reshape(1, D), W2)
    agg2 = _agg_kernel(hp2.reshape(NQ * N, DQ), src4, dst)
    return _tc_fin(agg2, hp2, dinv, b2.reshape(1, D))
